# baseline (speedup 1.0000x reference)
import jax
import jax.numpy as jnp
from jax.experimental import pallas as pl
from jax.experimental.pallas import tpu as pltpu

_B, _N = 32, 24
_NODE_F, _HID_F, _EDGE_F, _MSG, _OUT = 128, 128, 16, 128, 128
_PASSES = 3
_K = 8
_CH = _N // _K


def _mpnn_body(
    et_ref, nt_ref, wmsg_ref, wun_ref, wum_ref, wout_ref, out_ref,
    e3_ref, rowdeg_ref, actcnt_ref,
):
    k = pl.program_id(0)
    wm = wmsg_ref[...]
    w3 = wm[2 * _HID_F :]

    e4 = jnp.swapaxes(et_ref[...], 2, 3)
    e2 = e4.reshape(_CH * _N * _B, _EDGE_F)
    adj = jnp.sum(e2, axis=1, keepdims=True)
    edge_act = (adj != 0.0).astype(jnp.float32)
    rowdeg_ref[pl.ds(k * _CH, _CH)] = jnp.sum(adj.reshape(_CH, _N, _B, 1), axis=1)
    actcnt_ref[pl.ds(k * _CH, _CH)] = jnp.sum(edge_act.reshape(_CH, _N, _B, 1), axis=1)

    e3 = jnp.dot(e2, w3, preferred_element_type=jnp.float32)
    e3m = jnp.where(edge_act != 0.0, e3, -1e30)
    e3_ref[pl.ds(k * _CH, _CH)] = e3m.reshape(_CH, _N, _B, _MSG)

    @pl.when(k == _K - 1)
    def _passes():
        w1 = wm[:_HID_F]
        w2 = wm[_HID_F : 2 * _HID_F]
        node_act = rowdeg_ref[...].reshape(_N * _B, 1) != 0.0
        n_inact = _N - actcnt_ref[...].reshape(_N * _B, 1)

        hidden = jnp.swapaxes(nt_ref[...], 0, 1).reshape(_N * _B, _HID_F)
        for _ in range(_PASSES):
            a = jnp.dot(hidden, w1, preferred_element_type=jnp.float32)
            c = jnp.dot(hidden, w2, preferred_element_type=jnp.float32)
            a3 = a.reshape(_N, _B, _MSG)
            c3 = c.reshape(_N, _B, _MSG)
            acc = None
            for j in range(_N):
                term = jnp.tanh(e3_ref[:, j] + a3 + c3[j][None])
                acc = term if acc is None else acc + term
            msg = acc.reshape(_N * _B, _MSG) + n_inact
            upd = jnp.tanh(
                jnp.dot(hidden, wun_ref[...], preferred_element_type=jnp.float32)
                + jnp.dot(msg, wum_ref[...], preferred_element_type=jnp.float32)
            )
            hidden = jnp.where(node_act, upd, hidden)

        masked = jnp.where(node_act, hidden, 0.0)
        graph = jnp.sum(masked.reshape(_N, _B, _HID_F), axis=0)
        out_ref[...] = jnp.dot(graph, wout_ref[...], preferred_element_type=jnp.float32)


def kernel(nodes, edges, W_msg, W_un, W_um, W_out):
    et = edges.transpose(1, 2, 3, 0)

    return pl.pallas_call(
        _mpnn_body,
        grid=(_K,),
        in_specs=[
            pl.BlockSpec((_CH, _N, _EDGE_F, _B), lambda k: (k, 0, 0, 0)),
            pl.BlockSpec((_B, _N, _HID_F), lambda k: (0, 0, 0)),
            pl.BlockSpec((2 * _HID_F + _EDGE_F, _MSG), lambda k: (0, 0)),
            pl.BlockSpec((_HID_F, _HID_F), lambda k: (0, 0)),
            pl.BlockSpec((_MSG, _HID_F), lambda k: (0, 0)),
            pl.BlockSpec((_HID_F, _OUT), lambda k: (0, 0)),
        ],
        out_specs=pl.BlockSpec((_B, _OUT), lambda k: (0, 0)),
        out_shape=jax.ShapeDtypeStruct((_B, _OUT), jnp.float32),
        scratch_shapes=[
            pltpu.VMEM((_N, _N, _B, _MSG), jnp.float32),
            pltpu.VMEM((_N, _B, 1), jnp.float32),
            pltpu.VMEM((_N, _B, 1), jnp.float32),
        ],
        compiler_params=pltpu.CompilerParams(
            dimension_semantics=("arbitrary",),
        ),
    )(et, nodes, W_msg, W_un, W_um, W_out)


# speedup vs baseline: 1.1022x; 1.1022x over previous
import jax
import jax.numpy as jnp
from jax.experimental import pallas as pl
from jax.experimental.pallas import tpu as pltpu

_B, _N = 32, 24
_NODE_F, _HID_F, _EDGE_F, _MSG, _OUT = 128, 128, 16, 128, 128
_PASSES = 3
_K = 4
_CH = _N // _K


def _mpnn_body(
    et_ref, nt_ref, wmsg_ref, wun_ref, wum_ref, wout_ref, out_ref,
    e3_ref, rowdeg_ref, actcnt_ref,
):
    k = pl.program_id(0)
    wm = wmsg_ref[...]
    w3 = wm[2 * _HID_F :]

    e4 = jnp.swapaxes(et_ref[...], 2, 3)
    e2 = e4.reshape(_CH * _N * _B, _EDGE_F)
    adj = jnp.sum(e2, axis=1, keepdims=True)
    edge_act = (adj != 0.0).astype(jnp.float32)
    rowdeg_ref[pl.ds(k * _CH, _CH)] = jnp.sum(adj.reshape(_CH, _N, _B, 1), axis=1)
    actcnt_ref[pl.ds(k * _CH, _CH)] = jnp.sum(edge_act.reshape(_CH, _N, _B, 1), axis=1)

    e3 = jnp.dot(e2, w3, preferred_element_type=jnp.float32)
    e3m = jnp.where(edge_act != 0.0, e3, -1e30)
    e3_ref[pl.ds(k * _CH, _CH)] = e3m.reshape(_CH, _N, _B, _MSG)

    @pl.when(k == _K - 1)
    def _passes():
        w1 = wm[:_HID_F]
        w2 = wm[_HID_F : 2 * _HID_F]
        node_act = rowdeg_ref[...].reshape(_N * _B, 1) != 0.0
        n_inact = _N - actcnt_ref[...].reshape(_N * _B, 1)

        hidden = jnp.swapaxes(nt_ref[...], 0, 1).reshape(_N * _B, _HID_F)
        for _ in range(_PASSES):
            a = jnp.dot(hidden, w1, preferred_element_type=jnp.float32)
            c = jnp.dot(hidden, w2, preferred_element_type=jnp.float32)
            a3 = a.reshape(_N, _B, _MSG)
            c3 = c.reshape(_N, _B, _MSG)
            acc = None
            for j in range(_N):
                term = jnp.tanh(e3_ref[:, j] + a3 + c3[j][None])
                acc = term if acc is None else acc + term
            msg = acc.reshape(_N * _B, _MSG) + n_inact
            upd = jnp.tanh(
                jnp.dot(hidden, wun_ref[...], preferred_element_type=jnp.float32)
                + jnp.dot(msg, wum_ref[...], preferred_element_type=jnp.float32)
            )
            hidden = jnp.where(node_act, upd, hidden)

        masked = jnp.where(node_act, hidden, 0.0)
        graph = jnp.sum(masked.reshape(_N, _B, _HID_F), axis=0)
        out_ref[...] = jnp.dot(graph, wout_ref[...], preferred_element_type=jnp.float32)


def kernel(nodes, edges, W_msg, W_un, W_um, W_out):
    et = edges.transpose(1, 2, 3, 0)

    return pl.pallas_call(
        _mpnn_body,
        grid=(_K,),
        in_specs=[
            pl.BlockSpec((_CH, _N, _EDGE_F, _B), lambda k: (k, 0, 0, 0)),
            pl.BlockSpec((_B, _N, _HID_F), lambda k: (0, 0, 0)),
            pl.BlockSpec((2 * _HID_F + _EDGE_F, _MSG), lambda k: (0, 0)),
            pl.BlockSpec((_HID_F, _HID_F), lambda k: (0, 0)),
            pl.BlockSpec((_MSG, _HID_F), lambda k: (0, 0)),
            pl.BlockSpec((_HID_F, _OUT), lambda k: (0, 0)),
        ],
        out_specs=pl.BlockSpec((_B, _OUT), lambda k: (0, 0)),
        out_shape=jax.ShapeDtypeStruct((_B, _OUT), jnp.float32),
        scratch_shapes=[
            pltpu.VMEM((_N, _N, _B, _MSG), jnp.float32),
            pltpu.VMEM((_N, _B, 1), jnp.float32),
            pltpu.VMEM((_N, _B, 1), jnp.float32),
        ],
        compiler_params=pltpu.CompilerParams(
            dimension_semantics=("arbitrary",),
        ),
    )(et, nodes, W_msg, W_un, W_um, W_out)
